# Initial kernel scaffold; baseline (speedup 1.0000x reference)
import jax
import jax.numpy as jnp
from jax.experimental import pallas as pl


def kernel(x, edge_index, batch, W1, b1, preW, preb, postW, postb, linW, linb, bng, bnb, mW1, mb1, mW2, mb2, mW3, mb3):
    raise NotImplementedError("write your pallas kernel here")


# speedup vs baseline: 3.7839x; 3.7839x over previous
import functools

import jax
import jax.numpy as jnp
import numpy as np
from jax import lax
from jax.experimental import pallas as pl
from jax.experimental.pallas import tpu as pltpu
from jax.experimental.pallas import tpu_sc as plsc

N = 10000
E = 160000
G = 128
FT = 75
F = 80
NC, NS, LANES = 2, 16, 16
NW = NC * NS
NPT = 320
NPAD = NW * NPT
NB = 8
RB = NPAD // NB
K = 256
AVG_LOG = 2.8043990948177435
BIG = 3.0e38


def _sget(ref, i):
    return jnp.max(plsc.load_gather(ref, [jnp.full((LANES,), i, jnp.int32)]))


def _make_sc_reduce(full):
    nouts = 4 if full else 1
    outs = [jax.ShapeDtypeStruct((NPAD * F,), jnp.float32) for _ in range(nouts)]
    scratch = (
        [pltpu.VMEM((K,), jnp.int32),
         pltpu.VMEM((K,), jnp.int32),
         pltpu.VMEM((K, F), jnp.float32),
         pltpu.VMEM((40,), jnp.int32)]
        + [pltpu.VMEM((NPT * F,), jnp.float32) for _ in range(nouts)]
        + [pltpu.SemaphoreType.DMA]
    )
    mesh = plsc.VectorSubcoreMesh(core_axis_name="c", subcore_axis_name="s")

    @functools.partial(
        pl.kernel, out_type=outs, mesh=mesh, scratch_types=scratch,
        compiler_params=pltpu.CompilerParams(needs_layout_passes=False,
                                             use_tc_tiling_on_sc=False))
    def run(b_hbm, src_hbm, dloc_hbm, starts_hbm, *rest):
        out_hbm = rest[:nouts]
        sidx_v, dloc_v, rows_v, starts_v = rest[nouts:nouts + 4]
        accs = rest[nouts + 4:nouts + 4 + nouts]
        sem = rest[-1]
        w = lax.axis_index("s") * NC + lax.axis_index("c")

        zero = jnp.zeros((LANES,), jnp.float32)
        mn0 = jnp.full((LANES,), BIG, jnp.float32)

        def init_body(i, _):
            o = i * LANES
            accs[0][pl.ds(o, LANES)] = zero
            if full:
                accs[1][pl.ds(o, LANES)] = zero
                accs[2][pl.ds(o, LANES)] = mn0
                accs[3][pl.ds(o, LANES)] = -mn0
            return 0

        lax.fori_loop(0, NPT * F // LANES, init_body, 0)

        pltpu.sync_copy(starts_hbm, starts_v)
        s0 = _sget(starts_v, w)
        s1 = _sget(starts_v, w + 1)
        a0 = (s0 // 8) * 8
        nch = (s1 - a0 + K - 1) // K
        cidx = [lax.iota(jnp.int32, LANES) + fv * LANES for fv in range(F // LANES)]

        def chunk(c, _):
            ofs = a0 + c * K
            pltpu.sync_copy(src_hbm.at[pl.ds(ofs, K)], sidx_v)
            pltpu.sync_copy(dloc_hbm.at[pl.ds(ofs, K)], dloc_v)
            pltpu.async_copy(b_hbm.at[sidx_v], rows_v, sem).wait()
            lo = jnp.maximum(s0 - ofs, 0)
            hi = jnp.minimum(s1 - ofs, K)

            def edge(j, _):
                jv = jnp.full((LANES,), j, jnp.int32)
                dv = plsc.load_gather(dloc_v, [jv]) * F
                for fv in range(F // LANES):
                    af = dv + cidx[fv]
                    val = plsc.load_gather(rows_v, [jv, cidx[fv]])
                    plsc.addupdate_scatter(accs[0], [af], val)
                    if full:
                        plsc.addupdate_scatter(accs[1], [af], val * val)
                        m = plsc.load_gather(accs[2], [af])
                        plsc.store_scatter(accs[2], [af], jnp.minimum(m, val))
                        m = plsc.load_gather(accs[3], [af])
                        plsc.store_scatter(accs[3], [af], jnp.maximum(m, val))
                return 0

            lax.fori_loop(lo, hi, edge, 0)
            return 0

        lax.fori_loop(0, nch, chunk, 0)
        for t in range(nouts):
            pltpu.sync_copy(accs[t], out_hbm[t].at[pl.ds(w * NPT * F, NPT * F)])

    return run


_SC_CACHE = {}


def _sc_reduce(full):
    k = _SC_CACHE.get(full)
    if k is None:
        k = _make_sc_reduce(full)
        _SC_CACHE[full] = k
    return k


def _gcn_pre_body(x_ref, w1_ref, cnt_ref,
                  g_ref, dinv_ref, cs_ref, samp_ref, satt_ref):
    cnt = cnt_ref[...]
    dinv = lax.rsqrt(cnt + 1.0)
    cs = jnp.maximum(cnt, 1.0)
    lg = jnp.log(cs + 1.0)
    g_ref[...] = jnp.dot(x_ref[...], w1_ref[...],
                         preferred_element_type=jnp.float32, precision=lax.Precision.HIGHEST) * dinv
    dinv_ref[...] = dinv
    cs_ref[...] = cs
    samp_ref[...] = lg * (1.0 / AVG_LOG)
    satt_ref[...] = AVG_LOG / lg


def _gcn_fin_body(g_ref, gs_ref, dinv_ref, b1_ref, wd_ref, ws_ref, pb_ref,
                  h_ref, a_ref, b_ref):
    h = dinv_ref[...] * (gs_ref[...] + g_ref[...]) + b1_ref[...]
    h_ref[...] = h
    a_ref[...] = jnp.dot(h, wd_ref[...], preferred_element_type=jnp.float32, precision=lax.Precision.HIGHEST)
    b_ref[...] = jnp.dot(h, ws_ref[...],
                         preferred_element_type=jnp.float32, precision=lax.Precision.HIGHEST) + pb_ref[...]


def _post_body(h_ref, a_ref, s1_ref, s2_ref, mn_ref, mx_ref,
               cnt_ref, cs_ref, samp_ref, satt_ref,
               ph_ref, p1_ref, p2_ref, p3_ref, pb_ref, lw_ref, lb_ref,
               z_ref, zs_ref):
    A = a_ref[...]
    S1 = s1_ref[...]
    cnt = cnt_ref[...]
    cs = cs_ref[...]
    has = cnt > 0.0
    mb = S1 / cs
    mean = jnp.where(has, A + mb, 0.0)
    varb = s2_ref[...] / cs - mb * mb
    std = jnp.sqrt(jnp.maximum(varb, 0.0) + 1e-5)
    mn = jnp.where(has, A + mn_ref[...], 0.0)
    mx = jnp.where(has, A + mx_ref[...], 0.0)
    agg = jnp.concatenate([mean, mn, mx, std], axis=1)
    y = (jnp.dot(h_ref[...], ph_ref[...], preferred_element_type=jnp.float32, precision=lax.Precision.HIGHEST)
         + jnp.dot(agg, p1_ref[...], preferred_element_type=jnp.float32, precision=lax.Precision.HIGHEST)
         + samp_ref[...] * jnp.dot(agg, p2_ref[...],
                                   preferred_element_type=jnp.float32, precision=lax.Precision.HIGHEST)
         + satt_ref[...] * jnp.dot(agg, p3_ref[...],
                                   preferred_element_type=jnp.float32, precision=lax.Precision.HIGHEST)
         + pb_ref[...])
    z = jnp.dot(y, lw_ref[...], preferred_element_type=jnp.float32, precision=lax.Precision.HIGHEST) + lb_ref[...]
    rows = lax.broadcasted_iota(jnp.int32, (RB, 1), 0) + pl.program_id(0) * RB
    z = jnp.where(rows < N, z, 0.0)
    z_ref[...] = z
    zs_ref[...] = jnp.sum(z, axis=0).reshape(1, 1, F)


def _var_body(z_ref, zs_ref, vq_ref):
    m = jnp.sum(zs_ref[...], axis=0) * (1.0 / N)
    rows = lax.broadcasted_iota(jnp.int32, (RB, 1), 0) + pl.program_id(0) * RB
    d = jnp.where(rows < N, z_ref[...] - m, 0.0)
    vq_ref[...] = jnp.sum(d * d, axis=0).reshape(1, 1, F)


def _norm_pre_body(z_ref, zs_ref, zq_ref, bg_ref, bb_ref, wd_ref, ws_ref,
                   pb_ref, h_ref, a_ref, b_ref):
    m = jnp.sum(zs_ref[...], axis=0) * (1.0 / N)
    v = jnp.sum(zq_ref[...], axis=0) * (1.0 / N)
    hn = jnp.maximum((z_ref[...] - m) * lax.rsqrt(v + 1e-5) * bg_ref[...]
                     + bb_ref[...], 0.0)
    h_ref[...] = hn
    a_ref[...] = jnp.dot(hn, wd_ref[...], preferred_element_type=jnp.float32, precision=lax.Precision.HIGHEST)
    b_ref[...] = jnp.dot(hn, ws_ref[...],
                         preferred_element_type=jnp.float32, precision=lax.Precision.HIGHEST) + pb_ref[...]


def _norm_last_body(z_ref, zs_ref, zq_ref, bg_ref, bb_ref, h_ref):
    m = jnp.sum(zs_ref[...], axis=0) * (1.0 / N)
    v = jnp.sum(zq_ref[...], axis=0) * (1.0 / N)
    h_ref[...] = jnp.maximum((z_ref[...] - m) * lax.rsqrt(v + 1e-5) * bg_ref[...]
                             + bb_ref[...], 0.0)


def _pool_body(h_ref, batch_ref, w1_ref, b1_ref, w2_ref, b2_ref,
               w3_ref, b3_ref, out_ref):
    gids = lax.broadcasted_iota(jnp.int32, (NPAD, G), 1)
    onehot = (batch_ref[...] == gids).astype(jnp.float32)
    pooled = lax.dot_general(onehot, h_ref[...], (((0,), (0,)), ((), ())),
                             preferred_element_type=jnp.float32, precision=lax.Precision.HIGHEST)
    o = jnp.maximum(jnp.dot(pooled, w1_ref[...],
                            preferred_element_type=jnp.float32, precision=lax.Precision.HIGHEST) + b1_ref[...], 0.0)
    o = jnp.maximum(jnp.dot(o, w2_ref[...],
                            preferred_element_type=jnp.float32, precision=lax.Precision.HIGHEST) + b2_ref[...], 0.0)
    out_ref[...] = jnp.dot(o, w3_ref[...],
                           preferred_element_type=jnp.float32, precision=lax.Precision.HIGHEST) + b3_ref[...]


def _tc(body, outs, *args):
    return pl.pallas_call(body, out_shape=outs)(*args)


_BN = pl.BlockSpec((RB, F), lambda i: (i, 0))
_BC = pl.BlockSpec((RB, 1), lambda i: (i, 0))
_BP = pl.BlockSpec((1, 1, F), lambda i: (i, 0, 0))


def _rep(shape):
    nd = len(shape)
    return pl.BlockSpec(shape, lambda i: (0,) * nd)


def _f(shape):
    return jax.ShapeDtypeStruct(shape, jnp.float32)


def _pad2(a, r, c):
    return jnp.pad(a, ((0, r - a.shape[0]), (0, c - a.shape[1])))


def _pad_blocks(Wb):
    parts = [jnp.pad(Wb[j * FT:(j + 1) * FT], ((0, F - FT), (0, F - FT)))
             for j in range(4)]
    return jnp.concatenate(parts, axis=0)


def kernel(x, edge_index, batch, W1, b1, preW, preb, postW, postb,
           linW, linb, bng, bnb, mW1, mb1, mW2, mb2, mW3, mb3):
    i32 = jnp.int32
    src = edge_index[0].astype(i32)
    dst = edge_index[1].astype(i32)

    ks = jnp.sort(dst * 16384 + src)
    src_s = (ks & 16383).astype(i32)
    dst_s = (ks >> 14).astype(i32)
    dloc = (dst_s % NPT).astype(i32)
    b2 = jnp.searchsorted(dst_s, jnp.arange(N + 1, dtype=i32)).astype(i32)
    cnt = (b2[1:] - b2[:-1]).astype(jnp.float32)
    starts = b2[jnp.minimum(jnp.arange(NW + 1) * NPT, N)].astype(i32)
    starts_p = jnp.pad(starts, (0, 40 - (NW + 1)), constant_values=E)
    src_p = jnp.concatenate([src_s, jnp.zeros((K,), i32)])
    dloc_p = jnp.concatenate([dloc, jnp.zeros((K,), i32)])

    cnt_col = jnp.pad(cnt, (0, NPAD - N)).reshape(NPAD, 1)
    batch_col = jnp.pad(batch.astype(i32), (0, NPAD - N),
                        constant_values=-1).reshape(NPAD, 1)

    x_p = _pad2(x, NPAD, 8)
    W1_p = _pad2(W1, 8, F)
    b1_p = _pad2(b1.reshape(1, -1), 1, F)
    wd = [_pad2(preW[i, :FT], F, F) for i in range(5)]
    ws = [_pad2(preW[i, FT:], F, F) for i in range(5)]
    pb = [_pad2(preb[i].reshape(1, -1), 1, F) for i in range(5)]
    ph = [_pad2(postW[i, :FT], F, F) for i in range(5)]
    p1 = [_pad_blocks(postW[i, FT:FT + 300]) for i in range(5)]
    p2 = [_pad_blocks(postW[i, FT + 300:FT + 600]) for i in range(5)]
    p3 = [_pad_blocks(postW[i, FT + 600:FT + 900]) for i in range(5)]
    pob = [_pad2(postb[i].reshape(1, -1), 1, F) for i in range(5)]
    lw = [_pad2(linW[i], F, F) for i in range(5)]
    lb = [_pad2(linb[i].reshape(1, -1), 1, F) for i in range(5)]
    bg = [_pad2(bng[i].reshape(1, -1), 1, F) for i in range(5)]
    bb = [_pad2(bnb[i].reshape(1, -1), 1, F) for i in range(5)]
    mW1_p = _pad2(mW1, F, 56)
    mb1_p = _pad2(mb1.reshape(1, -1), 1, 56)
    mW2_p = _pad2(mW2, 56, 32)
    mb2_p = _pad2(mb2.reshape(1, -1), 1, 32)
    mW3_p = _pad2(mW3, 32, 16)
    mb3_p = _pad2(mb3.reshape(1, -1), 1, 16)

    col = _f((NPAD, 1))
    node = _f((NPAD, F))

    g, dinv, cs_col, samp, satt = pl.pallas_call(
        _gcn_pre_body, grid=(NB,),
        in_specs=[pl.BlockSpec((RB, 8), lambda i: (i, 0)), _rep((8, F)), _BC],
        out_specs=[_BN, _BC, _BC, _BC, _BC],
        out_shape=(node, col, col, col, col),
    )(x_p, W1_p, cnt_col)
    gs = _sc_reduce(False)(g, src_p, dloc_p, starts_p)[0].reshape(NPAD, F)
    h, A, B = pl.pallas_call(
        _gcn_fin_body, grid=(NB,),
        in_specs=[_BN, _BN, _BC, _rep((1, F)), _rep((F, F)), _rep((F, F)),
                  _rep((1, F))],
        out_specs=[_BN, _BN, _BN],
        out_shape=(node, node, node),
    )(g, gs, dinv, b1_p, wd[0], ws[0], pb[0])

    for i in range(5):
        s1f, s2f, mnf, mxf = _sc_reduce(True)(B, src_p, dloc_p, starts_p)
        S1 = s1f.reshape(NPAD, F)
        S2 = s2f.reshape(NPAD, F)
        MN = mnf.reshape(NPAD, F)
        MX = mxf.reshape(NPAD, F)
        z, zs = pl.pallas_call(
            _post_body, grid=(NB,),
            in_specs=[_BN] * 6 + [_BC] * 4 + [_rep((F, F))]
                     + [_rep((4 * F, F))] * 3
                     + [_rep((1, F)), _rep((F, F)), _rep((1, F))],
            out_specs=[_BN, _BP],
            out_shape=(node, _f((NB, 1, F))),
        )(h, A, S1, S2, MN, MX, cnt_col, cs_col, samp, satt,
          ph[i], p1[i], p2[i], p3[i], pob[i], lw[i], lb[i])
        zq = pl.pallas_call(
            _var_body, grid=(NB,),
            in_specs=[_BN, _rep((NB, 1, F))],
            out_specs=_BP, out_shape=_f((NB, 1, F)),
        )(z, zs)
        if i < 4:
            h, A, B = pl.pallas_call(
                _norm_pre_body, grid=(NB,),
                in_specs=[_BN, _rep((NB, 1, F)), _rep((NB, 1, F)),
                          _rep((1, F)), _rep((1, F)), _rep((F, F)),
                          _rep((F, F)), _rep((1, F))],
                out_specs=[_BN, _BN, _BN],
                out_shape=(node, node, node),
            )(z, zs, zq, bg[i], bb[i], wd[i + 1], ws[i + 1], pb[i + 1])
        else:
            h = pl.pallas_call(
                _norm_last_body, grid=(NB,),
                in_specs=[_BN, _rep((NB, 1, F)), _rep((NB, 1, F)),
                          _rep((1, F)), _rep((1, F))],
                out_specs=_BN, out_shape=node,
            )(z, zs, zq, bg[i], bb[i])

    out = _tc(_pool_body, (_f((G, 16)),),
              h, batch_col, mW1_p, mb1_p, mW2_p, mb2_p, mW3_p, mb3_p)[0]
    return out[:, :10]


# speedup vs baseline: 4.8188x; 1.2735x over previous
import functools

import jax
import jax.numpy as jnp
import numpy as np
from jax import lax
from jax.experimental import pallas as pl
from jax.experimental.pallas import tpu as pltpu
from jax.experimental.pallas import tpu_sc as plsc

N = 10000
E = 160000
G = 128
FT = 75
F = 80
NC, NS, LANES = 2, 16, 16
NW = NC * NS
NPT = 320
NPAD = NW * NPT
NB = 8
RB = NPAD // NB
K = 256
AVG_LOG = 2.8043990948177435
BIG = 3.0e38


def _sget(ref, i):
    return jnp.max(plsc.load_gather(ref, [jnp.full((LANES,), i, jnp.int32)]))


def _make_sc_reduce(full):
    nouts = 4 if full else 1
    outs = [jax.ShapeDtypeStruct((NPAD * F,), jnp.float32) for _ in range(nouts)]
    scratch = (
        [pltpu.VMEM((K,), jnp.int32),
         pltpu.VMEM((K,), jnp.int32),
         pltpu.VMEM((K, F), jnp.float32),
         pltpu.VMEM((40,), jnp.int32)]
        + [pltpu.VMEM(((NPT + 1) * F,), jnp.float32) for _ in range(nouts)]
        + [pltpu.SemaphoreType.DMA]
    )
    mesh = plsc.VectorSubcoreMesh(core_axis_name="c", subcore_axis_name="s")
    NV = F // LANES

    @functools.partial(
        pl.kernel, out_type=outs, mesh=mesh, scratch_types=scratch,
        compiler_params=pltpu.CompilerParams(needs_layout_passes=False,
                                             use_tc_tiling_on_sc=False))
    def run(b_hbm, src_hbm, wrow_hbm, starts_hbm, *rest):
        out_hbm = rest[:nouts]
        sidx_v, wrow_v, rows_v, starts_v = rest[nouts:nouts + 4]
        accs = rest[nouts + 4:nouts + 4 + nouts]
        sem = rest[-1]
        w = lax.axis_index("s") * NC + lax.axis_index("c")

        zero = jnp.zeros((LANES,), jnp.float32)
        big = jnp.full((LANES,), BIG, jnp.float32)

        def init_body(i, _):
            o = i * LANES
            accs[0][pl.ds(o, LANES)] = zero
            if full:
                accs[1][pl.ds(o, LANES)] = zero
                accs[2][pl.ds(o, LANES)] = big
                accs[3][pl.ds(o, LANES)] = -big
            return 0

        lax.fori_loop(0, (NPT + 1) * F // LANES, init_body, 0)

        pltpu.sync_copy(starts_hbm, starts_v)
        s0 = _sget(starts_v, w)
        s1 = _sget(starts_v, w + 1)
        a0 = (s0 // 8) * 8
        nch = (s1 - a0 + K - 1) // K
        cidx = [lax.iota(jnp.int32, LANES) + fv * LANES for fv in range(NV)]

        if full:
            regs0 = tuple([zero] * NV + [zero] * NV + [big] * NV + [-big] * NV)
        else:
            regs0 = tuple([zero] * NV)

        def chunk(c, regs):
            ofs = a0 + c * K
            pltpu.sync_copy(src_hbm.at[pl.ds(ofs, K)], sidx_v)
            pltpu.sync_copy(wrow_hbm.at[pl.ds(ofs, K)], wrow_v)
            pltpu.async_copy(b_hbm.at[sidx_v], rows_v, sem).wait()
            lo = jnp.maximum(s0 - ofs, 0)
            hi = jnp.minimum(s1 - ofs, K)

            def edge(j, regs):
                jv = jnp.full((LANES,), j, jnp.int32)
                av = plsc.load_gather(wrow_v, [jv]) * F
                fl = av < (NPT * F)
                regs = list(regs)
                for fv in range(NV):
                    af = av + cidx[fv]
                    val = plsc.load_gather(rows_v, [jv, cidx[fv]])
                    s = regs[fv] + val
                    plsc.store_scatter(accs[0], [af], s)
                    regs[fv] = jnp.where(fl, zero, s)
                    if full:
                        q = regs[NV + fv] + val * val
                        mn = jnp.minimum(regs[2 * NV + fv], val)
                        mx = jnp.maximum(regs[3 * NV + fv], val)
                        plsc.store_scatter(accs[1], [af], q)
                        plsc.store_scatter(accs[2], [af], mn)
                        plsc.store_scatter(accs[3], [af], mx)
                        regs[NV + fv] = jnp.where(fl, zero, q)
                        regs[2 * NV + fv] = jnp.where(fl, big, mn)
                        regs[3 * NV + fv] = jnp.where(fl, -big, mx)
                return tuple(regs)

            return lax.fori_loop(lo, hi, edge, regs)

        lax.fori_loop(0, nch, chunk, regs0)
        for t in range(nouts):
            pltpu.sync_copy(accs[t].at[pl.ds(0, NPT * F)],
                            out_hbm[t].at[pl.ds(w * NPT * F, NPT * F)])

    return run


_SC_CACHE = {}


def _sc_reduce(full):
    k = _SC_CACHE.get(full)
    if k is None:
        k = _make_sc_reduce(full)
        _SC_CACHE[full] = k
    return k


def _gcn_pre_body(x_ref, w1_ref, cnt_ref,
                  g_ref, dinv_ref, cs_ref, samp_ref, satt_ref):
    cnt = cnt_ref[...]
    dinv = lax.rsqrt(cnt + 1.0)
    cs = jnp.maximum(cnt, 1.0)
    lg = jnp.log(cs + 1.0)
    g_ref[...] = jnp.dot(x_ref[...], w1_ref[...],
                         preferred_element_type=jnp.float32, precision=lax.Precision.HIGHEST) * dinv
    dinv_ref[...] = dinv
    cs_ref[...] = cs
    samp_ref[...] = lg * (1.0 / AVG_LOG)
    satt_ref[...] = AVG_LOG / lg


def _gcn_fin_body(g_ref, gs_ref, dinv_ref, b1_ref, wd_ref, ws_ref, pb_ref,
                  h_ref, a_ref, b_ref):
    h = dinv_ref[...] * (gs_ref[...] + g_ref[...]) + b1_ref[...]
    h_ref[...] = h
    a_ref[...] = jnp.dot(h, wd_ref[...], preferred_element_type=jnp.float32, precision=lax.Precision.HIGHEST)
    b_ref[...] = jnp.dot(h, ws_ref[...],
                         preferred_element_type=jnp.float32, precision=lax.Precision.HIGHEST) + pb_ref[...]


def _post_body(h_ref, a_ref, s1_ref, s2_ref, mn_ref, mx_ref,
               cnt_ref, cs_ref, samp_ref, satt_ref,
               ph_ref, p1_ref, p2_ref, p3_ref, pb_ref, lw_ref, lb_ref,
               z_ref, zs_ref):
    A = a_ref[...]
    S1 = s1_ref[...]
    cnt = cnt_ref[...]
    cs = cs_ref[...]
    has = cnt > 0.0
    mb = S1 / cs
    mean = jnp.where(has, A + mb, 0.0)
    varb = s2_ref[...] / cs - mb * mb
    std = jnp.sqrt(jnp.maximum(varb, 0.0) + 1e-5)
    mn = jnp.where(has, A + mn_ref[...], 0.0)
    mx = jnp.where(has, A + mx_ref[...], 0.0)
    agg = jnp.concatenate([mean, mn, mx, std], axis=1)
    y = (jnp.dot(h_ref[...], ph_ref[...], preferred_element_type=jnp.float32, precision=lax.Precision.HIGHEST)
         + jnp.dot(agg, p1_ref[...], preferred_element_type=jnp.float32, precision=lax.Precision.HIGHEST)
         + samp_ref[...] * jnp.dot(agg, p2_ref[...],
                                   preferred_element_type=jnp.float32, precision=lax.Precision.HIGHEST)
         + satt_ref[...] * jnp.dot(agg, p3_ref[...],
                                   preferred_element_type=jnp.float32, precision=lax.Precision.HIGHEST)
         + pb_ref[...])
    z = jnp.dot(y, lw_ref[...], preferred_element_type=jnp.float32, precision=lax.Precision.HIGHEST) + lb_ref[...]
    rows = lax.broadcasted_iota(jnp.int32, (RB, 1), 0) + pl.program_id(0) * RB
    z = jnp.where(rows < N, z, 0.0)
    z_ref[...] = z
    zs_ref[...] = jnp.sum(z, axis=0).reshape(1, 1, F)


def _var_body(z_ref, zs_ref, vq_ref):
    m = jnp.sum(zs_ref[...], axis=0) * (1.0 / N)
    rows = lax.broadcasted_iota(jnp.int32, (RB, 1), 0) + pl.program_id(0) * RB
    d = jnp.where(rows < N, z_ref[...] - m, 0.0)
    vq_ref[...] = jnp.sum(d * d, axis=0).reshape(1, 1, F)


def _norm_pre_body(z_ref, zs_ref, zq_ref, bg_ref, bb_ref, wd_ref, ws_ref,
                   pb_ref, h_ref, a_ref, b_ref):
    m = jnp.sum(zs_ref[...], axis=0) * (1.0 / N)
    v = jnp.sum(zq_ref[...], axis=0) * (1.0 / N)
    hn = jnp.maximum((z_ref[...] - m) * lax.rsqrt(v + 1e-5) * bg_ref[...]
                     + bb_ref[...], 0.0)
    h_ref[...] = hn
    a_ref[...] = jnp.dot(hn, wd_ref[...], preferred_element_type=jnp.float32, precision=lax.Precision.HIGHEST)
    b_ref[...] = jnp.dot(hn, ws_ref[...],
                         preferred_element_type=jnp.float32, precision=lax.Precision.HIGHEST) + pb_ref[...]


def _norm_last_body(z_ref, zs_ref, zq_ref, bg_ref, bb_ref, h_ref):
    m = jnp.sum(zs_ref[...], axis=0) * (1.0 / N)
    v = jnp.sum(zq_ref[...], axis=0) * (1.0 / N)
    h_ref[...] = jnp.maximum((z_ref[...] - m) * lax.rsqrt(v + 1e-5) * bg_ref[...]
                             + bb_ref[...], 0.0)


def _pool_body(h_ref, batch_ref, w1_ref, b1_ref, w2_ref, b2_ref,
               w3_ref, b3_ref, out_ref):
    gids = lax.broadcasted_iota(jnp.int32, (NPAD, G), 1)
    onehot = (batch_ref[...] == gids).astype(jnp.float32)
    pooled = lax.dot_general(onehot, h_ref[...], (((0,), (0,)), ((), ())),
                             preferred_element_type=jnp.float32, precision=lax.Precision.HIGHEST)
    o = jnp.maximum(jnp.dot(pooled, w1_ref[...],
                            preferred_element_type=jnp.float32, precision=lax.Precision.HIGHEST) + b1_ref[...], 0.0)
    o = jnp.maximum(jnp.dot(o, w2_ref[...],
                            preferred_element_type=jnp.float32, precision=lax.Precision.HIGHEST) + b2_ref[...], 0.0)
    out_ref[...] = jnp.dot(o, w3_ref[...],
                           preferred_element_type=jnp.float32, precision=lax.Precision.HIGHEST) + b3_ref[...]


def _tc(body, outs, *args):
    return pl.pallas_call(body, out_shape=outs)(*args)


_BN = pl.BlockSpec((RB, F), lambda i: (i, 0))
_BC = pl.BlockSpec((RB, 1), lambda i: (i, 0))
_BP = pl.BlockSpec((1, 1, F), lambda i: (i, 0, 0))


def _rep(shape):
    nd = len(shape)
    return pl.BlockSpec(shape, lambda i: (0,) * nd)


def _f(shape):
    return jax.ShapeDtypeStruct(shape, jnp.float32)


def _pad2(a, r, c):
    return jnp.pad(a, ((0, r - a.shape[0]), (0, c - a.shape[1])))


def _pad_blocks(Wb):
    parts = [jnp.pad(Wb[j * FT:(j + 1) * FT], ((0, F - FT), (0, F - FT)))
             for j in range(4)]
    return jnp.concatenate(parts, axis=0)


def kernel(x, edge_index, batch, W1, b1, preW, preb, postW, postb,
           linW, linb, bng, bnb, mW1, mb1, mW2, mb2, mW3, mb3):
    i32 = jnp.int32
    src = edge_index[0].astype(i32)
    dst = edge_index[1].astype(i32)

    ks = jnp.sort(dst * 16384 + src)
    src_s = (ks & 16383).astype(i32)
    dst_s = (ks >> 14).astype(i32)
    nxt = jnp.concatenate([dst_s[1:], jnp.full((1,), -1, i32)])
    wrow = jnp.where(dst_s != nxt, dst_s % NPT, NPT).astype(i32)
    b2 = jnp.searchsorted(dst_s, jnp.arange(N + 1, dtype=i32)).astype(i32)
    cnt = (b2[1:] - b2[:-1]).astype(jnp.float32)
    starts = b2[jnp.minimum(jnp.arange(NW + 1) * NPT, N)].astype(i32)
    starts_p = jnp.pad(starts, (0, 40 - (NW + 1)), constant_values=E)
    src_p = jnp.concatenate([src_s, jnp.zeros((K,), i32)])
    dloc_p = jnp.concatenate([wrow, jnp.full((K,), NPT, i32)])

    cnt_col = jnp.pad(cnt, (0, NPAD - N)).reshape(NPAD, 1)
    batch_col = jnp.pad(batch.astype(i32), (0, NPAD - N),
                        constant_values=-1).reshape(NPAD, 1)

    x_p = _pad2(x, NPAD, 8)
    W1_p = _pad2(W1, 8, F)
    b1_p = _pad2(b1.reshape(1, -1), 1, F)
    wd = [_pad2(preW[i, :FT], F, F) for i in range(5)]
    ws = [_pad2(preW[i, FT:], F, F) for i in range(5)]
    pb = [_pad2(preb[i].reshape(1, -1), 1, F) for i in range(5)]
    ph = [_pad2(postW[i, :FT], F, F) for i in range(5)]
    p1 = [_pad_blocks(postW[i, FT:FT + 300]) for i in range(5)]
    p2 = [_pad_blocks(postW[i, FT + 300:FT + 600]) for i in range(5)]
    p3 = [_pad_blocks(postW[i, FT + 600:FT + 900]) for i in range(5)]
    pob = [_pad2(postb[i].reshape(1, -1), 1, F) for i in range(5)]
    lw = [_pad2(linW[i], F, F) for i in range(5)]
    lb = [_pad2(linb[i].reshape(1, -1), 1, F) for i in range(5)]
    bg = [_pad2(bng[i].reshape(1, -1), 1, F) for i in range(5)]
    bb = [_pad2(bnb[i].reshape(1, -1), 1, F) for i in range(5)]
    mW1_p = _pad2(mW1, F, 56)
    mb1_p = _pad2(mb1.reshape(1, -1), 1, 56)
    mW2_p = _pad2(mW2, 56, 32)
    mb2_p = _pad2(mb2.reshape(1, -1), 1, 32)
    mW3_p = _pad2(mW3, 32, 16)
    mb3_p = _pad2(mb3.reshape(1, -1), 1, 16)

    col = _f((NPAD, 1))
    node = _f((NPAD, F))

    g, dinv, cs_col, samp, satt = pl.pallas_call(
        _gcn_pre_body, grid=(NB,),
        in_specs=[pl.BlockSpec((RB, 8), lambda i: (i, 0)), _rep((8, F)), _BC],
        out_specs=[_BN, _BC, _BC, _BC, _BC],
        out_shape=(node, col, col, col, col),
    )(x_p, W1_p, cnt_col)
    gs = _sc_reduce(False)(g, src_p, dloc_p, starts_p)[0].reshape(NPAD, F)
    h, A, B = pl.pallas_call(
        _gcn_fin_body, grid=(NB,),
        in_specs=[_BN, _BN, _BC, _rep((1, F)), _rep((F, F)), _rep((F, F)),
                  _rep((1, F))],
        out_specs=[_BN, _BN, _BN],
        out_shape=(node, node, node),
    )(g, gs, dinv, b1_p, wd[0], ws[0], pb[0])

    for i in range(5):
        s1f, s2f, mnf, mxf = _sc_reduce(True)(B, src_p, dloc_p, starts_p)
        S1 = s1f.reshape(NPAD, F)
        S2 = s2f.reshape(NPAD, F)
        MN = mnf.reshape(NPAD, F)
        MX = mxf.reshape(NPAD, F)
        z, zs = pl.pallas_call(
            _post_body, grid=(NB,),
            in_specs=[_BN] * 6 + [_BC] * 4 + [_rep((F, F))]
                     + [_rep((4 * F, F))] * 3
                     + [_rep((1, F)), _rep((F, F)), _rep((1, F))],
            out_specs=[_BN, _BP],
            out_shape=(node, _f((NB, 1, F))),
        )(h, A, S1, S2, MN, MX, cnt_col, cs_col, samp, satt,
          ph[i], p1[i], p2[i], p3[i], pob[i], lw[i], lb[i])
        zq = pl.pallas_call(
            _var_body, grid=(NB,),
            in_specs=[_BN, _rep((NB, 1, F))],
            out_specs=_BP, out_shape=_f((NB, 1, F)),
        )(z, zs)
        if i < 4:
            h, A, B = pl.pallas_call(
                _norm_pre_body, grid=(NB,),
                in_specs=[_BN, _rep((NB, 1, F)), _rep((NB, 1, F)),
                          _rep((1, F)), _rep((1, F)), _rep((F, F)),
                          _rep((F, F)), _rep((1, F))],
                out_specs=[_BN, _BN, _BN],
                out_shape=(node, node, node),
            )(z, zs, zq, bg[i], bb[i], wd[i + 1], ws[i + 1], pb[i + 1])
        else:
            h = pl.pallas_call(
                _norm_last_body, grid=(NB,),
                in_specs=[_BN, _rep((NB, 1, F)), _rep((NB, 1, F)),
                          _rep((1, F)), _rep((1, F))],
                out_specs=_BN, out_shape=node,
            )(z, zs, zq, bg[i], bb[i])

    out = _tc(_pool_body, (_f((G, 16)),),
              h, batch_col, mW1_p, mb1_p, mW2_p, mb2_p, mW3_p, mb3_p)[0]
    return out[:, :10]


# speedup vs baseline: 4.9444x; 1.0261x over previous
import functools

import jax
import jax.numpy as jnp
import numpy as np
from jax import lax
from jax.experimental import pallas as pl
from jax.experimental.pallas import tpu as pltpu
from jax.experimental.pallas import tpu_sc as plsc

N = 10000
E = 160000
G = 128
FT = 75
F = 80
NC, NS, LANES = 2, 16, 16
NW = NC * NS
NPT = 320
NPAD = NW * NPT
NB = 8
RB = NPAD // NB
K = 128
AVG_LOG = 2.8043990948177435
BIG = 3.0e38


def _sget(ref, i):
    return jnp.max(plsc.load_gather(ref, [jnp.full((LANES,), i, jnp.int32)]))


def _make_sc_reduce(full):
    nouts = 4 if full else 1
    outs = [jax.ShapeDtypeStruct((NPAD * F,), jnp.float32) for _ in range(nouts)]
    scratch = (
        [pltpu.VMEM((K,), jnp.int32), pltpu.VMEM((K,), jnp.int32),
         pltpu.VMEM((K,), jnp.int32), pltpu.VMEM((K,), jnp.int32),
         pltpu.VMEM((K, F), jnp.float32), pltpu.VMEM((K, F), jnp.float32),
         pltpu.VMEM((40,), jnp.int32)]
        + [pltpu.VMEM(((NPT + 1) * F,), jnp.float32) for _ in range(nouts)]
        + [pltpu.SemaphoreType.DMA, pltpu.SemaphoreType.DMA]
    )
    mesh = plsc.VectorSubcoreMesh(core_axis_name="c", subcore_axis_name="s")
    NV = F // LANES

    @functools.partial(
        pl.kernel, out_type=outs, mesh=mesh, scratch_types=scratch,
        compiler_params=pltpu.CompilerParams(needs_layout_passes=False,
                                             use_tc_tiling_on_sc=False))
    def run(b_hbm, src_hbm, wrow_hbm, starts_hbm, *rest):
        out_hbm = rest[:nouts]
        sidx = rest[nouts:nouts + 2]
        wrows = rest[nouts + 2:nouts + 4]
        rows = rest[nouts + 4:nouts + 6]
        starts_v = rest[nouts + 6]
        accs = rest[nouts + 7:nouts + 7 + nouts]
        sems = rest[-2:]
        w = lax.axis_index("s") * NC + lax.axis_index("c")

        zero = jnp.zeros((LANES,), jnp.float32)
        big = jnp.full((LANES,), BIG, jnp.float32)

        def init_body(i, _):
            o = i * LANES
            accs[0][pl.ds(o, LANES)] = zero
            if full:
                accs[1][pl.ds(o, LANES)] = zero
                accs[2][pl.ds(o, LANES)] = big
                accs[3][pl.ds(o, LANES)] = -big
            return 0

        lax.fori_loop(0, (NPT + 1) * F // LANES, init_body, 0)

        pltpu.sync_copy(starts_hbm, starts_v)
        s0 = _sget(starts_v, w)
        s1 = _sget(starts_v, w + 1)
        a0 = (s0 // 8) * 8
        nch = (s1 - a0 + K - 1) // K
        nch2 = ((jnp.maximum(nch, 1) + 1) // 2) * 2
        cidx = [lax.iota(jnp.int32, LANES) + fv * LANES for fv in range(NV)]

        if full:
            regs0 = tuple([zero] * NV + [zero] * NV + [big] * NV + [-big] * NV)
        else:
            regs0 = tuple([zero] * NV)

        def issue(c, buf):
            ofs = a0 + c * K
            pltpu.sync_copy(src_hbm.at[pl.ds(ofs, K)], sidx[buf])
            pltpu.sync_copy(wrow_hbm.at[pl.ds(ofs, K)], wrows[buf])
            pltpu.async_copy(b_hbm.at[sidx[buf]], rows[buf], sems[buf])

        def process(c, buf, regs):
            ofs = a0 + c * K
            lo = jnp.maximum(s0 - ofs, 0)
            hi = jnp.minimum(s1 - ofs, K)

            def edge(j, regs):
                jv = jnp.full((LANES,), j, jnp.int32)
                av = plsc.load_gather(wrows[buf], [jv]) * F
                fl = av < (NPT * F)
                regs = list(regs)
                for fv in range(NV):
                    af = av + cidx[fv]
                    val = plsc.load_gather(rows[buf], [jv, cidx[fv]])
                    s = regs[fv] + val
                    plsc.store_scatter(accs[0], [af], s)
                    regs[fv] = jnp.where(fl, zero, s)
                    if full:
                        q = regs[NV + fv] + val * val
                        mn = jnp.minimum(regs[2 * NV + fv], val)
                        mx = jnp.maximum(regs[3 * NV + fv], val)
                        plsc.store_scatter(accs[1], [af], q)
                        plsc.store_scatter(accs[2], [af], mn)
                        plsc.store_scatter(accs[3], [af], mx)
                        regs[NV + fv] = jnp.where(fl, zero, q)
                        regs[2 * NV + fv] = jnp.where(fl, big, mn)
                        regs[3 * NV + fv] = jnp.where(fl, -big, mx)
                return tuple(regs)

            return lax.fori_loop(lo, hi, edge, regs)

        issue(0, 0)

        def pair(c0, regs):
            c = 2 * c0

            @pl.when(c + 1 < nch2)
            def _():
                issue(c + 1, 1)

            pltpu.make_async_copy(b_hbm.at[sidx[0]], rows[0], sems[0]).wait()
            regs = process(c, 0, regs)

            @pl.when(c + 2 < nch2)
            def _():
                issue(c + 2, 0)

            pltpu.make_async_copy(b_hbm.at[sidx[1]], rows[1], sems[1]).wait()
            return process(c + 1, 1, regs)

        lax.fori_loop(0, nch2 // 2, pair, regs0)
        for t in range(nouts):
            pltpu.sync_copy(accs[t].at[pl.ds(0, NPT * F)],
                            out_hbm[t].at[pl.ds(w * NPT * F, NPT * F)])

    return run


_SC_CACHE = {}


def _sc_reduce(full):
    k = _SC_CACHE.get(full)
    if k is None:
        k = _make_sc_reduce(full)
        _SC_CACHE[full] = k
    return k


def _gcn_pre_body(x_ref, w1_ref, cnt_ref,
                  g_ref, dinv_ref, cs_ref, samp_ref, satt_ref):
    cnt = cnt_ref[...]
    dinv = lax.rsqrt(cnt + 1.0)
    cs = jnp.maximum(cnt, 1.0)
    lg = jnp.log(cs + 1.0)
    g_ref[...] = jnp.dot(x_ref[...], w1_ref[...],
                         preferred_element_type=jnp.float32, precision=lax.Precision.HIGHEST) * dinv
    dinv_ref[...] = dinv
    cs_ref[...] = cs
    samp_ref[...] = lg * (1.0 / AVG_LOG)
    satt_ref[...] = AVG_LOG / lg


def _gcn_fin_body(g_ref, gs_ref, dinv_ref, b1_ref, wd_ref, ws_ref, pb_ref,
                  h_ref, a_ref, b_ref):
    h = dinv_ref[...] * (gs_ref[...] + g_ref[...]) + b1_ref[...]
    h_ref[...] = h
    a_ref[...] = jnp.dot(h, wd_ref[...], preferred_element_type=jnp.float32, precision=lax.Precision.HIGHEST)
    b_ref[...] = jnp.dot(h, ws_ref[...],
                         preferred_element_type=jnp.float32, precision=lax.Precision.HIGHEST) + pb_ref[...]


def _post_body(h_ref, a_ref, s1_ref, s2_ref, mn_ref, mx_ref,
               cnt_ref, cs_ref, samp_ref, satt_ref,
               ph_ref, p1_ref, p2_ref, p3_ref, pb_ref, lw_ref, lb_ref,
               z_ref, zs_ref):
    A = a_ref[...]
    S1 = s1_ref[...]
    cnt = cnt_ref[...]
    cs = cs_ref[...]
    has = cnt > 0.0
    mb = S1 / cs
    mean = jnp.where(has, A + mb, 0.0)
    varb = s2_ref[...] / cs - mb * mb
    std = jnp.sqrt(jnp.maximum(varb, 0.0) + 1e-5)
    mn = jnp.where(has, A + mn_ref[...], 0.0)
    mx = jnp.where(has, A + mx_ref[...], 0.0)
    agg = jnp.concatenate([mean, mn, mx, std], axis=1)
    y = (jnp.dot(h_ref[...], ph_ref[...], preferred_element_type=jnp.float32, precision=lax.Precision.HIGHEST)
         + jnp.dot(agg, p1_ref[...], preferred_element_type=jnp.float32, precision=lax.Precision.HIGHEST)
         + samp_ref[...] * jnp.dot(agg, p2_ref[...],
                                   preferred_element_type=jnp.float32, precision=lax.Precision.HIGHEST)
         + satt_ref[...] * jnp.dot(agg, p3_ref[...],
                                   preferred_element_type=jnp.float32, precision=lax.Precision.HIGHEST)
         + pb_ref[...])
    z = jnp.dot(y, lw_ref[...], preferred_element_type=jnp.float32, precision=lax.Precision.HIGHEST) + lb_ref[...]
    rows = lax.broadcasted_iota(jnp.int32, (RB, 1), 0) + pl.program_id(0) * RB
    z = jnp.where(rows < N, z, 0.0)
    z_ref[...] = z
    zs_ref[...] = jnp.sum(z, axis=0).reshape(1, 1, F)


def _var_body(z_ref, zs_ref, vq_ref):
    m = jnp.sum(zs_ref[...], axis=0) * (1.0 / N)
    rows = lax.broadcasted_iota(jnp.int32, (RB, 1), 0) + pl.program_id(0) * RB
    d = jnp.where(rows < N, z_ref[...] - m, 0.0)
    vq_ref[...] = jnp.sum(d * d, axis=0).reshape(1, 1, F)


def _norm_pre_body(z_ref, zs_ref, zq_ref, bg_ref, bb_ref, wd_ref, ws_ref,
                   pb_ref, h_ref, a_ref, b_ref):
    m = jnp.sum(zs_ref[...], axis=0) * (1.0 / N)
    v = jnp.sum(zq_ref[...], axis=0) * (1.0 / N)
    hn = jnp.maximum((z_ref[...] - m) * lax.rsqrt(v + 1e-5) * bg_ref[...]
                     + bb_ref[...], 0.0)
    h_ref[...] = hn
    a_ref[...] = jnp.dot(hn, wd_ref[...], preferred_element_type=jnp.float32, precision=lax.Precision.HIGHEST)
    b_ref[...] = jnp.dot(hn, ws_ref[...],
                         preferred_element_type=jnp.float32, precision=lax.Precision.HIGHEST) + pb_ref[...]


def _norm_last_body(z_ref, zs_ref, zq_ref, bg_ref, bb_ref, h_ref):
    m = jnp.sum(zs_ref[...], axis=0) * (1.0 / N)
    v = jnp.sum(zq_ref[...], axis=0) * (1.0 / N)
    h_ref[...] = jnp.maximum((z_ref[...] - m) * lax.rsqrt(v + 1e-5) * bg_ref[...]
                             + bb_ref[...], 0.0)


def _pool_body(h_ref, batch_ref, w1_ref, b1_ref, w2_ref, b2_ref,
               w3_ref, b3_ref, out_ref):
    gids = lax.broadcasted_iota(jnp.int32, (NPAD, G), 1)
    onehot = (batch_ref[...] == gids).astype(jnp.float32)
    pooled = lax.dot_general(onehot, h_ref[...], (((0,), (0,)), ((), ())),
                             preferred_element_type=jnp.float32, precision=lax.Precision.HIGHEST)
    o = jnp.maximum(jnp.dot(pooled, w1_ref[...],
                            preferred_element_type=jnp.float32, precision=lax.Precision.HIGHEST) + b1_ref[...], 0.0)
    o = jnp.maximum(jnp.dot(o, w2_ref[...],
                            preferred_element_type=jnp.float32, precision=lax.Precision.HIGHEST) + b2_ref[...], 0.0)
    out_ref[...] = jnp.dot(o, w3_ref[...],
                           preferred_element_type=jnp.float32, precision=lax.Precision.HIGHEST) + b3_ref[...]


def _tc(body, outs, *args):
    return pl.pallas_call(body, out_shape=outs)(*args)


_BN = pl.BlockSpec((RB, F), lambda i: (i, 0))
_BC = pl.BlockSpec((RB, 1), lambda i: (i, 0))
_BP = pl.BlockSpec((1, 1, F), lambda i: (i, 0, 0))


def _rep(shape):
    nd = len(shape)
    return pl.BlockSpec(shape, lambda i: (0,) * nd)


def _f(shape):
    return jax.ShapeDtypeStruct(shape, jnp.float32)


def _pad2(a, r, c):
    return jnp.pad(a, ((0, r - a.shape[0]), (0, c - a.shape[1])))


def _pad_blocks(Wb):
    parts = [jnp.pad(Wb[j * FT:(j + 1) * FT], ((0, F - FT), (0, F - FT)))
             for j in range(4)]
    return jnp.concatenate(parts, axis=0)


def kernel(x, edge_index, batch, W1, b1, preW, preb, postW, postb,
           linW, linb, bng, bnb, mW1, mb1, mW2, mb2, mW3, mb3):
    i32 = jnp.int32
    src = edge_index[0].astype(i32)
    dst = edge_index[1].astype(i32)

    ks = jnp.sort(dst * 16384 + src)
    src_s = (ks & 16383).astype(i32)
    dst_s = (ks >> 14).astype(i32)
    nxt = jnp.concatenate([dst_s[1:], jnp.full((1,), -1, i32)])
    wrow = jnp.where(dst_s != nxt, dst_s % NPT, NPT).astype(i32)
    b2 = jnp.searchsorted(dst_s, jnp.arange(N + 1, dtype=i32)).astype(i32)
    cnt = (b2[1:] - b2[:-1]).astype(jnp.float32)
    starts = b2[jnp.minimum(jnp.arange(NW + 1) * NPT, N)].astype(i32)
    starts_p = jnp.pad(starts, (0, 40 - (NW + 1)), constant_values=E)
    src_p = jnp.concatenate([src_s, jnp.zeros((4 * K,), i32)])
    dloc_p = jnp.concatenate([wrow, jnp.full((4 * K,), NPT, i32)])

    cnt_col = jnp.pad(cnt, (0, NPAD - N)).reshape(NPAD, 1)
    batch_col = jnp.pad(batch.astype(i32), (0, NPAD - N),
                        constant_values=-1).reshape(NPAD, 1)

    x_p = _pad2(x, NPAD, 8)
    W1_p = _pad2(W1, 8, F)
    b1_p = _pad2(b1.reshape(1, -1), 1, F)
    wd = [_pad2(preW[i, :FT], F, F) for i in range(5)]
    ws = [_pad2(preW[i, FT:], F, F) for i in range(5)]
    pb = [_pad2(preb[i].reshape(1, -1), 1, F) for i in range(5)]
    ph = [_pad2(postW[i, :FT], F, F) for i in range(5)]
    p1 = [_pad_blocks(postW[i, FT:FT + 300]) for i in range(5)]
    p2 = [_pad_blocks(postW[i, FT + 300:FT + 600]) for i in range(5)]
    p3 = [_pad_blocks(postW[i, FT + 600:FT + 900]) for i in range(5)]
    pob = [_pad2(postb[i].reshape(1, -1), 1, F) for i in range(5)]
    lw = [_pad2(linW[i], F, F) for i in range(5)]
    lb = [_pad2(linb[i].reshape(1, -1), 1, F) for i in range(5)]
    bg = [_pad2(bng[i].reshape(1, -1), 1, F) for i in range(5)]
    bb = [_pad2(bnb[i].reshape(1, -1), 1, F) for i in range(5)]
    mW1_p = _pad2(mW1, F, 56)
    mb1_p = _pad2(mb1.reshape(1, -1), 1, 56)
    mW2_p = _pad2(mW2, 56, 32)
    mb2_p = _pad2(mb2.reshape(1, -1), 1, 32)
    mW3_p = _pad2(mW3, 32, 16)
    mb3_p = _pad2(mb3.reshape(1, -1), 1, 16)

    col = _f((NPAD, 1))
    node = _f((NPAD, F))

    g, dinv, cs_col, samp, satt = pl.pallas_call(
        _gcn_pre_body, grid=(NB,),
        in_specs=[pl.BlockSpec((RB, 8), lambda i: (i, 0)), _rep((8, F)), _BC],
        out_specs=[_BN, _BC, _BC, _BC, _BC],
        out_shape=(node, col, col, col, col),
    )(x_p, W1_p, cnt_col)
    gs = _sc_reduce(False)(g, src_p, dloc_p, starts_p)[0].reshape(NPAD, F)
    h, A, B = pl.pallas_call(
        _gcn_fin_body, grid=(NB,),
        in_specs=[_BN, _BN, _BC, _rep((1, F)), _rep((F, F)), _rep((F, F)),
                  _rep((1, F))],
        out_specs=[_BN, _BN, _BN],
        out_shape=(node, node, node),
    )(g, gs, dinv, b1_p, wd[0], ws[0], pb[0])

    for i in range(5):
        s1f, s2f, mnf, mxf = _sc_reduce(True)(B, src_p, dloc_p, starts_p)
        S1 = s1f.reshape(NPAD, F)
        S2 = s2f.reshape(NPAD, F)
        MN = mnf.reshape(NPAD, F)
        MX = mxf.reshape(NPAD, F)
        z, zs = pl.pallas_call(
            _post_body, grid=(NB,),
            in_specs=[_BN] * 6 + [_BC] * 4 + [_rep((F, F))]
                     + [_rep((4 * F, F))] * 3
                     + [_rep((1, F)), _rep((F, F)), _rep((1, F))],
            out_specs=[_BN, _BP],
            out_shape=(node, _f((NB, 1, F))),
        )(h, A, S1, S2, MN, MX, cnt_col, cs_col, samp, satt,
          ph[i], p1[i], p2[i], p3[i], pob[i], lw[i], lb[i])
        zq = pl.pallas_call(
            _var_body, grid=(NB,),
            in_specs=[_BN, _rep((NB, 1, F))],
            out_specs=_BP, out_shape=_f((NB, 1, F)),
        )(z, zs)
        if i < 4:
            h, A, B = pl.pallas_call(
                _norm_pre_body, grid=(NB,),
                in_specs=[_BN, _rep((NB, 1, F)), _rep((NB, 1, F)),
                          _rep((1, F)), _rep((1, F)), _rep((F, F)),
                          _rep((F, F)), _rep((1, F))],
                out_specs=[_BN, _BN, _BN],
                out_shape=(node, node, node),
            )(z, zs, zq, bg[i], bb[i], wd[i + 1], ws[i + 1], pb[i + 1])
        else:
            h = pl.pallas_call(
                _norm_last_body, grid=(NB,),
                in_specs=[_BN, _rep((NB, 1, F)), _rep((NB, 1, F)),
                          _rep((1, F)), _rep((1, F))],
                out_specs=_BN, out_shape=node,
            )(z, zs, zq, bg[i], bb[i])

    out = _tc(_pool_body, (_f((G, 16)),),
              h, batch_col, mW1_p, mb1_p, mW2_p, mb2_p, mW3_p, mb3_p)[0]
    return out[:, :10]
